# Initial kernel scaffold; baseline (speedup 1.0000x reference)
import jax
import jax.numpy as jnp
from jax.experimental import pallas as pl


def kernel(hidden_states, ln_w, ln_b, router_w, w_gate, w_up, w_down):
    raise NotImplementedError("write your pallas kernel here")


# speedup vs baseline: 1.3618x; 1.3618x over previous
import functools

import jax
import jax.numpy as jnp
from jax.experimental import pallas as pl
from jax.experimental.pallas import tpu as pltpu

EPS = 1e-5
LANES = 128


def _ln_router_body(hs_ref, lnw_ref, lnb_ref, rw_ref, xhat_ref, comb_ref):
    x = hs_ref[...]
    mu = jnp.mean(x, axis=-1, keepdims=True)
    var = jnp.mean((x - mu) ** 2, axis=-1, keepdims=True)
    xhat = (x - mu) * jax.lax.rsqrt(var + EPS)
    xhat = xhat * lnw_ref[...] + lnb_ref[...]
    xhat_ref[...] = xhat

    logits = jnp.dot(xhat, rw_ref[...], preferred_element_type=jnp.float32)
    iota = jax.lax.broadcasted_iota(jnp.int32, logits.shape, 1)
    valid = iota < 8
    neg = jnp.float32(-1e30)
    logits = jnp.where(valid, logits, neg)
    m = jnp.max(logits, axis=-1, keepdims=True)
    ex = jnp.where(valid, jnp.exp(logits - m), 0.0)
    scores = ex / jnp.sum(ex, axis=-1, keepdims=True)
    s = jnp.where(valid, scores, neg)
    m1 = jnp.max(s, axis=-1, keepdims=True)
    i1 = jnp.min(jnp.where(s == m1, iota, 128), axis=-1, keepdims=True)
    s2 = jnp.where(iota == i1, neg, s)
    m2 = jnp.max(s2, axis=-1, keepdims=True)
    i2 = jnp.min(jnp.where(s2 == m2, iota, 128), axis=-1, keepdims=True)
    comb = jnp.where(iota == i1, m1, 0.0) + jnp.where(iota == i2, m2, 0.0)
    comb_ref[...] = comb


def _moe_dense_body(x_ref, comb_ref, wg_ref, wu_ref, wd_ref, out_ref):
    e = pl.program_id(1)
    nf = pl.program_id(2)

    @pl.when((e == 0) & (nf == 0))
    def _init():
        out_ref[...] = jnp.zeros_like(out_ref)

    x = x_ref[...]
    g = jnp.dot(x, wg_ref[0], preferred_element_type=jnp.float32)
    u = jnp.dot(x, wu_ref[0], preferred_element_type=jnp.float32)
    h = (g * jax.nn.sigmoid(g)) * u
    y = jnp.dot(h, wd_ref[0], preferred_element_type=jnp.float32)
    comb = comb_ref[...]
    iota = jax.lax.broadcasted_iota(jnp.int32, comb.shape, 1)
    cvec = jnp.sum(jnp.where(iota == e, comb, 0.0), axis=-1, keepdims=True)
    out_ref[...] += cvec * y


def kernel(hidden_states, ln_w, ln_b, router_w, w_gate, w_up, w_down):
    T, D = hidden_states.shape
    E = router_w.shape[1]
    F = w_gate.shape[2]

    BT1 = min(256, T)
    rw_pad = jnp.zeros((D, LANES), jnp.float32).at[:, :E].set(router_w)

    xhat, comb = pl.pallas_call(
        _ln_router_body,
        grid=(T // BT1,),
        in_specs=[
            pl.BlockSpec((BT1, D), lambda t: (t, 0)),
            pl.BlockSpec((1, D), lambda t: (0, 0)),
            pl.BlockSpec((1, D), lambda t: (0, 0)),
            pl.BlockSpec((D, LANES), lambda t: (0, 0)),
        ],
        out_specs=[
            pl.BlockSpec((BT1, D), lambda t: (t, 0)),
            pl.BlockSpec((BT1, LANES), lambda t: (t, 0)),
        ],
        out_shape=[
            jax.ShapeDtypeStruct((T, D), jnp.float32),
            jax.ShapeDtypeStruct((T, LANES), jnp.float32),
        ],
    )(hidden_states, ln_w.reshape(1, D), ln_b.reshape(1, D), rw_pad)

    BT2 = min(512, T)
    BF = 1408 if F % 1408 == 0 else F
    NT, NF = T // BT2, F // BF

    out = pl.pallas_call(
        _moe_dense_body,
        grid=(NT, E, NF),
        in_specs=[
            pl.BlockSpec((BT2, D), lambda t, e, nf: (t, 0)),
            pl.BlockSpec((BT2, LANES), lambda t, e, nf: (t, 0)),
            pl.BlockSpec((1, D, BF), lambda t, e, nf: (e, 0, nf)),
            pl.BlockSpec((1, D, BF), lambda t, e, nf: (e, 0, nf)),
            pl.BlockSpec((1, BF, D), lambda t, e, nf: (e, nf, 0)),
        ],
        out_specs=pl.BlockSpec((BT2, D), lambda t, e, nf: (t, 0)),
        out_shape=jax.ShapeDtypeStruct((T, D), jnp.float32),
    )(xhat, comb, w_gate, w_up, w_down)
    return out
